# baseline (speedup 1.0000x reference)
import functools

import jax
import jax.numpy as jnp
from jax import lax
from jax.experimental import pallas as pl
from jax.experimental.pallas import tpu as pltpu
from jax.experimental.pallas import tpu_sc as plsc

N = 10000
E = 320000
NP = 10240
NT = 16
NC = 2
NW = NC * NT
RPT = NP // NT
C = 128
CH = E // C
CPW = 78
TAILW = CH - NW * CPW


def _sc_mesh():
    return plsc.VectorSubcoreMesh(core_axis_name="c", subcore_axis_name="s")


_SC_PARAMS = pltpu.CompilerParams(use_tc_tiling_on_sc=False)


_DEG_K = 13

@functools.partial(
    pl.kernel,
    mesh=_sc_mesh(),
    compiler_params=_SC_PARAMS,
    out_type=[jax.ShapeDtypeStruct((NP,), jnp.float32),
              jax.ShapeDtypeStruct((NP,), jnp.float32)],
    scratch_types=[
        pltpu.VMEM((_DEG_K, C), jnp.int32),
        pltpu.VMEM((C,), jnp.float32),
        pltpu.VMEM_SHARED((NP,), jnp.float32),
        pltpu.SemaphoreType.DMA,
        pltpu.SemaphoreType.DMA,
    ],
)
def _sc_degree(dst2_hbm, zeros_hbm, out_a_hbm, out_b_hbm, didx, ones_v, acc,
               isem, ssem):
    cid = lax.axis_index("c")
    sid = lax.axis_index("s")
    wid = cid * NT + sid
    r0 = sid * RPT
    for i in range(C // 16):
        ones_v[pl.ds(i * 16, 16)] = jnp.ones((16,), jnp.float32)
    pltpu.sync_copy(zeros_hbm.at[pl.ds(r0, RPT)], acc.at[pl.ds(r0, RPT)])
    plsc.subcore_barrier()

    def body(blk, carry):
        row0 = wid * CPW + blk * _DEG_K
        pltpu.async_copy(dst2_hbm.at[pl.ds(row0, _DEG_K)], didx, isem).wait()
        ss = [pltpu.async_copy(ones_v, acc.at[didx.at[k]], ssem, add=True)
              for k in range(_DEG_K)]
        for s in ss:
            s.wait()
        return carry

    lax.fori_loop(0, CPW // _DEG_K, body, 0)

    @pl.when(wid < TAILW)
    def _():
        trow = NW * CPW + wid
        pltpu.async_copy(dst2_hbm.at[pl.ds(trow, 1)],
                         didx.at[pl.ds(0, 1)], isem).wait()
        pltpu.async_copy(ones_v, acc.at[didx.at[0]], ssem, add=True).wait()

    plsc.subcore_barrier()
    @pl.when(cid == 0)
    def _():
        pltpu.sync_copy(acc.at[pl.ds(r0, RPT)], out_a_hbm.at[pl.ds(r0, RPT)])
    @pl.when(cid == 1)
    def _():
        pltpu.sync_copy(acc.at[pl.ds(r0, RPT)], out_b_hbm.at[pl.ds(r0, RPT)])


def _make_sc_agg(F, K):
    NBLK = CPW // K

    @functools.partial(
        pl.kernel,
        mesh=_sc_mesh(),
        compiler_params=_SC_PARAMS,
        out_type=[jax.ShapeDtypeStruct((NP, F), jnp.float32),
                  jax.ShapeDtypeStruct((NP, F), jnp.float32)],
        scratch_types=[
            pltpu.VMEM((3, K, C), jnp.int32),
            pltpu.VMEM((3, K, C), jnp.int32),
            pltpu.VMEM((2, K, C, F), jnp.float32),
            pltpu.VMEM_SHARED((NP, F), jnp.float32),
            pltpu.SemaphoreType.DMA,
            pltpu.SemaphoreType.DMA,
            pltpu.SemaphoreType.DMA,
            pltpu.SemaphoreType.DMA,
            pltpu.SemaphoreType.DMA,
        ],
    )
    def _sc_agg(h_hbm, src2_hbm, dst2_hbm, zeros_hbm, out_a_hbm, out_b_hbm,
                sidx, didx, rows, acc, is0, is1, is2, gsem, ssem):
        cid = lax.axis_index("c")
        sid = lax.axis_index("s")
        wid = cid * NT + sid
        r0 = sid * RPT
        pltpu.sync_copy(zeros_hbm.at[pl.ds(r0, RPT)], acc.at[pl.ds(r0, RPT)])
        plsc.subcore_barrier()
        base = wid * CPW
        isems = [is0, is1, is2]

        def fire_idx(b):
            sl = b % 3
            r = base + b * K
            return (pltpu.async_copy(src2_hbm.at[pl.ds(r, K)], sidx.at[sl],
                                     isems[sl]),
                    pltpu.async_copy(dst2_hbm.at[pl.ds(r, K)], didx.at[sl],
                                     isems[sl]))

        def fire_g(b):
            return [pltpu.async_copy(h_hbm.at[sidx.at[b % 3, k]],
                                     rows.at[b % 2, k], gsem)
                    for k in range(K)]

        def fire_s(b):
            return [pltpu.async_copy(rows.at[b % 2, k],
                                     acc.at[didx.at[b % 3, k]], ssem,
                                     add=True)
                    for k in range(K)]

        idxh, gh, sh = {}, {}, {}
        for b in range(min(3, NBLK)):
            idxh[b] = fire_idx(b)
        for h in idxh[0]:
            h.wait()
        gh[0] = fire_g(0)
        for h in gh[0]:
            h.wait()
        sh[0] = fire_s(0)
        for h in idxh[1]:
            h.wait()
        gh[1] = fire_g(1)
        for b in range(2, NBLK):
            for h in sh[b - 2]:
                h.wait()
            if b + 1 < NBLK:
                idxh[b + 1] = fire_idx(b + 1)
            for h in gh[b - 1]:
                h.wait()
            sh[b - 1] = fire_s(b - 1)
            for h in idxh[b]:
                h.wait()
            gh[b] = fire_g(b)
        for h in gh[NBLK - 1]:
            h.wait()
        sh[NBLK - 1] = fire_s(NBLK - 1)
        for h in sh[NBLK - 2]:
            h.wait()
        for h in sh[NBLK - 1]:
            h.wait()

        @pl.when(wid < TAILW)
        def _():
            trow = NW * CPW + wid
            i1 = pltpu.async_copy(src2_hbm.at[pl.ds(trow, 1)],
                                  sidx.at[0, pl.ds(0, 1)], is0)
            i2 = pltpu.async_copy(dst2_hbm.at[pl.ds(trow, 1)],
                                  didx.at[0, pl.ds(0, 1)], is0)
            i1.wait()
            i2.wait()
            pltpu.async_copy(h_hbm.at[sidx.at[0, 0]], rows.at[0, 0],
                             gsem).wait()
            pltpu.async_copy(rows.at[0, 0], acc.at[didx.at[0, 0]], ssem,
                             add=True).wait()

        plsc.subcore_barrier()
        @pl.when(cid == 0)
        def _():
            pltpu.sync_copy(acc.at[pl.ds(r0, RPT)],
                            out_a_hbm.at[pl.ds(r0, RPT)])
        @pl.when(cid == 1)
        def _():
            pltpu.sync_copy(acc.at[pl.ds(r0, RPT)],
                            out_b_hbm.at[pl.ds(r0, RPT)])

    return _sc_agg


_sc_agg32 = _make_sc_agg(32, 6)
_sc_agg64 = _make_sc_agg(64, 3)


def _dinv(dega_ref, degb_ref):
    deg = dega_ref[0:N, :] + degb_ref[0:N, :] + 1.0
    return lax.rsqrt(deg)


def _tc_a(x_ref, w1_ref, dega_ref, degb_ref, h1p_ref):
    dinv = _dinv(dega_ref, degb_ref)
    h1 = jnp.dot(x_ref[...], w1_ref[...], preferred_element_type=jnp.float32)
    h1p_ref[...] = h1 * dinv


def _tc_b(agg_a_ref, agg_b_ref, h1p_ref, dega_ref, degb_ref, b1_ref,
          bnw_ref, bnb_ref, w2_ref, h2p_ref):
    dinv = _dinv(dega_ref, degb_ref)
    aggsum = agg_a_ref[0:N, :] + agg_b_ref[0:N, :]
    out1 = dinv * (aggsum + h1p_ref[...]) + b1_ref[...]
    inv_n = jnp.float32(1.0 / N)
    mean = jnp.sum(out1, axis=0, keepdims=True) * inv_n
    cent = out1 - mean
    var = jnp.sum(cent * cent, axis=0, keepdims=True) * inv_n
    z = cent * lax.rsqrt(var + 1e-5) * bnw_ref[...] + bnb_ref[...]
    z = jnp.maximum(z, 0.0)
    h2 = jnp.dot(z, w2_ref[...], preferred_element_type=jnp.float32)
    h2p_ref[...] = h2 * dinv


def _tc_c(agg_a_ref, agg_b_ref, h2p_ref, dega_ref, degb_ref, b2_ref,
          out_ref):
    dinv = _dinv(dega_ref, degb_ref)
    aggsum = agg_a_ref[0:N, :] + agg_b_ref[0:N, :]
    o = dinv * (aggsum + h2p_ref[...]) + b2_ref[...]
    m = jnp.max(o, axis=1, keepdims=True)
    e = jnp.exp(o - m)
    lse = jnp.log(jnp.sum(e, axis=1, keepdims=True))
    out_ref[...] = o - m - lse


def _tc_call(body, out_shape, *args):
    return pl.pallas_call(
        body,
        out_shape=jax.ShapeDtypeStruct(out_shape, jnp.float32),
    )(*args)


def kernel(x, edge_index, W1, b1, bn_w, bn_b, W2, b2):
    f32 = jnp.float32
    src2 = jnp.reshape(edge_index[0], (CH, C))
    dst2 = jnp.reshape(edge_index[1], (CH, C))
    zeros_n = jnp.zeros((NP,), f32)
    zeros_32 = jnp.zeros((NP, 32), f32)
    zeros_64 = jnp.zeros((NP, 64), f32)

    dega, degb = _sc_degree(dst2, zeros_n)
    da = jnp.reshape(dega, (NP, 1))
    db = jnp.reshape(degb, (NP, 1))

    h1p = _tc_call(_tc_a, (N, 32), x, W1, da, db)

    agg1a, agg1b = _sc_agg32(h1p, src2, dst2, zeros_32)

    h2p = _tc_call(_tc_b, (N, 64),
                   agg1a, agg1b, h1p, da, db,
                   jnp.reshape(b1, (1, 32)), jnp.reshape(bn_w, (1, 32)),
                   jnp.reshape(bn_b, (1, 32)), W2)

    agg2a, agg2b = _sc_agg64(h2p, src2, dst2, zeros_64)

    return _tc_call(_tc_c, (N, 64),
                    agg2a, agg2b, h2p, da, db,
                    jnp.reshape(b2, (1, 64)))


# speedup vs baseline: 1.0253x; 1.0253x over previous
import functools

import jax
import jax.numpy as jnp
from jax import lax
from jax.experimental import pallas as pl
from jax.experimental.pallas import tpu as pltpu
from jax.experimental.pallas import tpu_sc as plsc

N = 10000
E = 320000
NP = 10240
NT = 16
NC = 2
NW = NC * NT
RPT = NP // NT
C = 128
CPW = 80
EPW = CPW * C
EP = NW * EPW


def _sc_mesh():
    return plsc.VectorSubcoreMesh(core_axis_name="c", subcore_axis_name="s")


_SC_PARAMS = pltpu.CompilerParams(use_tc_tiling_on_sc=False)


_DEG_K = 16

@functools.partial(
    pl.kernel,
    mesh=_sc_mesh(),
    compiler_params=_SC_PARAMS,
    out_type=[jax.ShapeDtypeStruct((NP,), jnp.float32),
              jax.ShapeDtypeStruct((NP,), jnp.float32)],
    scratch_types=[
        pltpu.VMEM((_DEG_K, C), jnp.int32),
        pltpu.VMEM((C,), jnp.float32),
        pltpu.VMEM_SHARED((NP,), jnp.float32),
        pltpu.SemaphoreType.DMA,
        pltpu.SemaphoreType.DMA,
    ],
)
def _sc_degree(dst2_hbm, zeros_hbm, out_a_hbm, out_b_hbm, didx, ones_v, acc,
               isem, ssem):
    cid = lax.axis_index("c")
    sid = lax.axis_index("s")
    wid = cid * NT + sid
    r0 = sid * RPT
    for i in range(C // 16):
        ones_v[pl.ds(i * 16, 16)] = jnp.ones((16,), jnp.float32)
    pltpu.sync_copy(zeros_hbm.at[pl.ds(r0, RPT)], acc.at[pl.ds(r0, RPT)])
    plsc.subcore_barrier()

    def body(blk, carry):
        row0 = pl.multiple_of(wid * CPW + blk * _DEG_K, _DEG_K)
        pltpu.async_copy(dst2_hbm.at[pl.ds(row0, _DEG_K)], didx, isem).wait()
        ss = [pltpu.async_copy(ones_v, acc.at[didx.at[k]], ssem, add=True)
              for k in range(_DEG_K)]
        for s in ss:
            s.wait()
        return carry

    lax.fori_loop(0, CPW // _DEG_K, body, 0)
    plsc.subcore_barrier()
    @pl.when(cid == 0)
    def _():
        pltpu.sync_copy(acc.at[pl.ds(r0, RPT)], out_a_hbm.at[pl.ds(r0, RPT)])
    @pl.when(cid == 1)
    def _():
        pltpu.sync_copy(acc.at[pl.ds(r0, RPT)], out_b_hbm.at[pl.ds(r0, RPT)])


def _make_sc_agg(F, K):
    NBLK = CPW // K

    @functools.partial(
        pl.kernel,
        mesh=_sc_mesh(),
        compiler_params=_SC_PARAMS,
        out_type=[jax.ShapeDtypeStruct((NP, F), jnp.float32),
                  jax.ShapeDtypeStruct((NP, F), jnp.float32)],
        scratch_types=[
            pltpu.VMEM((3, K, C), jnp.int32),
            pltpu.VMEM((3, K, C), jnp.int32),
            pltpu.VMEM((2, K, C, F), jnp.float32),
            pltpu.VMEM_SHARED((NP, F), jnp.float32),
            pltpu.SemaphoreType.DMA,
            pltpu.SemaphoreType.DMA,
            pltpu.SemaphoreType.DMA,
            pltpu.SemaphoreType.DMA,
            pltpu.SemaphoreType.DMA,
        ],
    )
    def _sc_agg(h_hbm, src2_hbm, dst2_hbm, zeros_hbm, out_a_hbm, out_b_hbm,
                sidx, didx, rows, acc, is0, is1, is2, gsem, ssem):
        cid = lax.axis_index("c")
        sid = lax.axis_index("s")
        wid = cid * NT + sid
        r0 = sid * RPT
        pltpu.sync_copy(zeros_hbm.at[pl.ds(r0, RPT)], acc.at[pl.ds(r0, RPT)])
        plsc.subcore_barrier()
        base = wid * CPW
        isems = [is0, is1, is2]

        def fire_idx(b):
            sl = b % 3
            r = pl.multiple_of(base + b * K, K)
            return (pltpu.async_copy(src2_hbm.at[pl.ds(r, K)], sidx.at[sl],
                                     isems[sl]),
                    pltpu.async_copy(dst2_hbm.at[pl.ds(r, K)], didx.at[sl],
                                     isems[sl]))

        def fire_g(b):
            return [pltpu.async_copy(h_hbm.at[sidx.at[b % 3, k]],
                                     rows.at[b % 2, k], gsem)
                    for k in range(K)]

        def fire_s(b):
            return [pltpu.async_copy(rows.at[b % 2, k],
                                     acc.at[didx.at[b % 3, k]], ssem,
                                     add=True)
                    for k in range(K)]

        idxh, gh, sh = {}, {}, {}
        for b in range(min(3, NBLK)):
            idxh[b] = fire_idx(b)
        for h in idxh[0]:
            h.wait()
        gh[0] = fire_g(0)
        for h in gh[0]:
            h.wait()
        sh[0] = fire_s(0)
        for h in idxh[1]:
            h.wait()
        gh[1] = fire_g(1)
        for b in range(2, NBLK):
            for h in sh[b - 2]:
                h.wait()
            if b + 1 < NBLK:
                idxh[b + 1] = fire_idx(b + 1)
            for h in gh[b - 1]:
                h.wait()
            sh[b - 1] = fire_s(b - 1)
            for h in idxh[b]:
                h.wait()
            gh[b] = fire_g(b)
        for h in gh[NBLK - 1]:
            h.wait()
        sh[NBLK - 1] = fire_s(NBLK - 1)
        for h in sh[NBLK - 2]:
            h.wait()
        for h in sh[NBLK - 1]:
            h.wait()
        plsc.subcore_barrier()
        @pl.when(cid == 0)
        def _():
            pltpu.sync_copy(acc.at[pl.ds(r0, RPT)],
                            out_a_hbm.at[pl.ds(r0, RPT)])
        @pl.when(cid == 1)
        def _():
            pltpu.sync_copy(acc.at[pl.ds(r0, RPT)],
                            out_b_hbm.at[pl.ds(r0, RPT)])

    return _sc_agg


_sc_agg32 = _make_sc_agg(32, 8)
_sc_agg64 = _make_sc_agg(64, 4)


def _dinv_col(degcols_ref):
    deg = degcols_ref[:, 0:1] + degcols_ref[:, 1:2] + 1.0
    dinv = lax.rsqrt(deg)
    rows = lax.broadcasted_iota(jnp.int32, (NP, 1), 0)
    return jnp.where(rows < N, dinv, 0.0)


def _tc_a(x_ref, w1_ref, degc_ref, h1p_ref):
    dinv = _dinv_col(degc_ref)
    h1 = jnp.dot(x_ref[...], w1_ref[...], preferred_element_type=jnp.float32)
    h1p_ref[0:N, :] = h1 * dinv[0:N, :]
    h1p_ref[N:NP, :] = jnp.zeros((NP - N, 32), jnp.float32)


def _tc_b(agg_a_ref, agg_b_ref, h1p_ref, degc_ref, b1_ref, bnw_ref, bnb_ref,
          w2_ref, h2p_ref):
    dinv = _dinv_col(degc_ref)
    aggsum = agg_a_ref[...] + agg_b_ref[...]
    out1 = dinv * (aggsum + h1p_ref[...]) + b1_ref[...]
    rows = lax.broadcasted_iota(jnp.int32, (NP, 1), 0)
    mask = rows < N
    inv_n = jnp.float32(1.0 / N)
    mean = jnp.sum(jnp.where(mask, out1, 0.0), axis=0, keepdims=True) * inv_n
    cent = out1 - mean
    var = jnp.sum(jnp.where(mask, cent * cent, 0.0), axis=0,
                  keepdims=True) * inv_n
    z = cent * lax.rsqrt(var + 1e-5) * bnw_ref[...] + bnb_ref[...]
    z = jnp.maximum(z, 0.0)
    h2 = jnp.dot(z, w2_ref[...], preferred_element_type=jnp.float32)
    h2p_ref[...] = h2 * dinv


def _tc_c(agg_a_ref, agg_b_ref, h2p_ref, degc_ref, b2_ref, out_ref):
    dinv = _dinv_col(degc_ref)
    aggsum = agg_a_ref[...] + agg_b_ref[...]
    o = (dinv * (aggsum + h2p_ref[...]) + b2_ref[...])[0:N, :]
    m = jnp.max(o, axis=1, keepdims=True)
    e = jnp.exp(o - m)
    lse = jnp.log(jnp.sum(e, axis=1, keepdims=True))
    out_ref[...] = o - m - lse


def _tc_call(body, out_shape, *args):
    return pl.pallas_call(
        body,
        out_shape=jax.ShapeDtypeStruct(out_shape, jnp.float32),
    )(*args)


def kernel(x, edge_index, W1, b1, bn_w, bn_b, W2, b2):
    f32 = jnp.float32
    pad_idx = (jnp.arange(EP - E, dtype=jnp.int32) % (NP - N)) + N
    src = jnp.reshape(jnp.concatenate([edge_index[0], pad_idx]), (EP // C, C))
    dst = jnp.reshape(jnp.concatenate([edge_index[1], pad_idx]), (EP // C, C))
    zeros_n = jnp.zeros((NP,), f32)
    zeros_32 = jnp.zeros((NP, 32), f32)
    zeros_64 = jnp.zeros((NP, 64), f32)

    dega, degb = _sc_degree(dst, zeros_n)
    degcols = jnp.stack([dega, degb], axis=1)

    h1p = _tc_call(_tc_a, (NP, 32), x, W1, degcols)

    agg1a, agg1b = _sc_agg32(h1p, src, dst, zeros_32)

    h2p = _tc_call(_tc_b, (NP, 64),
                   agg1a, agg1b, h1p, degcols,
                   jnp.reshape(b1, (1, 32)), jnp.reshape(bn_w, (1, 32)),
                   jnp.reshape(bn_b, (1, 32)), W2)

    agg2a, agg2b = _sc_agg64(h2p, src, dst, zeros_64)

    return _tc_call(_tc_c, (N, 64),
                    agg2a, agg2b, h2p, degcols,
                    jnp.reshape(b2, (1, 64)))


# speedup vs baseline: 1.0320x; 1.0065x over previous
import functools

import jax
import jax.numpy as jnp
from jax import lax
from jax.experimental import pallas as pl
from jax.experimental.pallas import tpu as pltpu
from jax.experimental.pallas import tpu_sc as plsc

N = 10000
E = 320000
NP = 10240
NT = 16
NC = 2
NW = NC * NT
RPT = NP // NT
C = 128
CPW = 80
EPW = CPW * C
EP = NW * EPW


def _sc_mesh():
    return plsc.VectorSubcoreMesh(core_axis_name="c", subcore_axis_name="s")


_SC_PARAMS = pltpu.CompilerParams(use_tc_tiling_on_sc=False)


_DEG_K = 16

@functools.partial(
    pl.kernel,
    mesh=_sc_mesh(),
    compiler_params=_SC_PARAMS,
    out_type=[jax.ShapeDtypeStruct((NP,), jnp.float32),
              jax.ShapeDtypeStruct((NP,), jnp.float32)],
    scratch_types=[
        pltpu.VMEM((_DEG_K, C), jnp.int32),
        pltpu.VMEM((C,), jnp.float32),
        pltpu.VMEM_SHARED((NP,), jnp.float32),
        pltpu.SemaphoreType.DMA,
        pltpu.SemaphoreType.DMA,
    ],
)
def _sc_degree(dst2_hbm, zeros_hbm, out_a_hbm, out_b_hbm, didx, ones_v, acc,
               isem, ssem):
    cid = lax.axis_index("c")
    sid = lax.axis_index("s")
    wid = cid * NT + sid
    r0 = sid * RPT
    for i in range(C // 16):
        ones_v[pl.ds(i * 16, 16)] = jnp.ones((16,), jnp.float32)
    pltpu.sync_copy(zeros_hbm.at[pl.ds(r0, RPT)], acc.at[pl.ds(r0, RPT)])
    plsc.subcore_barrier()

    def body(blk, carry):
        row0 = pl.multiple_of(wid * CPW + blk * _DEG_K, _DEG_K)
        pltpu.async_copy(dst2_hbm.at[pl.ds(row0, _DEG_K)], didx, isem).wait()
        ss = [pltpu.async_copy(ones_v, acc.at[didx.at[k]], ssem, add=True)
              for k in range(_DEG_K)]
        for s in ss:
            s.wait()
        return carry

    lax.fori_loop(0, CPW // _DEG_K, body, 0)
    plsc.subcore_barrier()
    @pl.when(cid == 0)
    def _():
        pltpu.sync_copy(acc.at[pl.ds(r0, RPT)], out_a_hbm.at[pl.ds(r0, RPT)])
    @pl.when(cid == 1)
    def _():
        pltpu.sync_copy(acc.at[pl.ds(r0, RPT)], out_b_hbm.at[pl.ds(r0, RPT)])


def _make_sc_agg(F, K):
    NBLK = CPW // K

    @functools.partial(
        pl.kernel,
        mesh=_sc_mesh(),
        compiler_params=_SC_PARAMS,
        out_type=[jax.ShapeDtypeStruct((NP, F), jnp.float32),
                  jax.ShapeDtypeStruct((NP, F), jnp.float32)],
        scratch_types=[
            pltpu.VMEM((3, K, C), jnp.int32),
            pltpu.VMEM((3, K, C), jnp.int32),
            pltpu.VMEM((2, K, C, F), jnp.float32),
            pltpu.VMEM_SHARED((NP, F), jnp.float32),
            pltpu.SemaphoreType.DMA,
            pltpu.SemaphoreType.DMA,
            pltpu.SemaphoreType.DMA,
            pltpu.SemaphoreType.DMA,
            pltpu.SemaphoreType.DMA,
        ],
    )
    def _sc_agg(h_hbm, src2_hbm, dst2_hbm, zeros_hbm, out_a_hbm, out_b_hbm,
                sidx, didx, rows, acc, is0, is1, is2, gsem, ssem):
        cid = lax.axis_index("c")
        sid = lax.axis_index("s")
        wid = cid * NT + sid
        r0 = sid * RPT
        pltpu.sync_copy(zeros_hbm.at[pl.ds(r0, RPT)], acc.at[pl.ds(r0, RPT)])
        plsc.subcore_barrier()
        base = wid * CPW
        isems = [is0, is1, is2]

        def fire_idx(b):
            sl = b % 3
            r = pl.multiple_of(base + b * K, K)
            return (pltpu.async_copy(src2_hbm.at[pl.ds(r, K)], sidx.at[sl],
                                     isems[sl]),
                    pltpu.async_copy(dst2_hbm.at[pl.ds(r, K)], didx.at[sl],
                                     isems[sl]))

        def fire_g(b):
            return [pltpu.async_copy(h_hbm.at[sidx.at[b % 3, k]],
                                     rows.at[b % 2, k], gsem)
                    for k in range(K)]

        def fire_s(b):
            return [pltpu.async_copy(rows.at[b % 2, k],
                                     acc.at[didx.at[b % 3, k]], ssem,
                                     add=True)
                    for k in range(K)]

        idxh, gh, sh = {}, {}, {}
        for b in range(min(3, NBLK)):
            idxh[b] = fire_idx(b)
        for h in idxh[0]:
            h.wait()
        gh[0] = fire_g(0)
        for h in gh[0]:
            h.wait()
        sh[0] = fire_s(0)
        for h in idxh[1]:
            h.wait()
        gh[1] = fire_g(1)
        for b in range(2, NBLK):
            for h in sh[b - 2]:
                h.wait()
            if b + 1 < NBLK:
                idxh[b + 1] = fire_idx(b + 1)
            for h in gh[b - 1]:
                h.wait()
            sh[b - 1] = fire_s(b - 1)
            for h in idxh[b]:
                h.wait()
            gh[b] = fire_g(b)
        for h in gh[NBLK - 1]:
            h.wait()
        sh[NBLK - 1] = fire_s(NBLK - 1)
        for h in sh[NBLK - 2]:
            h.wait()
        for h in sh[NBLK - 1]:
            h.wait()
        plsc.subcore_barrier()
        @pl.when(cid == 0)
        def _():
            pltpu.sync_copy(acc.at[pl.ds(r0, RPT)],
                            out_a_hbm.at[pl.ds(r0, RPT)])
        @pl.when(cid == 1)
        def _():
            pltpu.sync_copy(acc.at[pl.ds(r0, RPT)],
                            out_b_hbm.at[pl.ds(r0, RPT)])

    return _sc_agg


_sc_agg32 = _make_sc_agg(32, 10)
_sc_agg64 = _make_sc_agg(64, 5)


def _dinv_col(degcols_ref):
    deg = degcols_ref[:, 0:1] + degcols_ref[:, 1:2] + 1.0
    dinv = lax.rsqrt(deg)
    rows = lax.broadcasted_iota(jnp.int32, (NP, 1), 0)
    return jnp.where(rows < N, dinv, 0.0)


def _tc_a(x_ref, w1_ref, degc_ref, h1p_ref):
    dinv = _dinv_col(degc_ref)
    h1 = jnp.dot(x_ref[...], w1_ref[...], preferred_element_type=jnp.float32)
    h1p_ref[0:N, :] = h1 * dinv[0:N, :]
    h1p_ref[N:NP, :] = jnp.zeros((NP - N, 32), jnp.float32)


def _tc_b(agg_a_ref, agg_b_ref, h1p_ref, degc_ref, b1_ref, bnw_ref, bnb_ref,
          w2_ref, h2p_ref):
    dinv = _dinv_col(degc_ref)
    aggsum = agg_a_ref[...] + agg_b_ref[...]
    out1 = dinv * (aggsum + h1p_ref[...]) + b1_ref[...]
    rows = lax.broadcasted_iota(jnp.int32, (NP, 1), 0)
    mask = rows < N
    inv_n = jnp.float32(1.0 / N)
    mean = jnp.sum(jnp.where(mask, out1, 0.0), axis=0, keepdims=True) * inv_n
    cent = out1 - mean
    var = jnp.sum(jnp.where(mask, cent * cent, 0.0), axis=0,
                  keepdims=True) * inv_n
    z = cent * lax.rsqrt(var + 1e-5) * bnw_ref[...] + bnb_ref[...]
    z = jnp.maximum(z, 0.0)
    h2 = jnp.dot(z, w2_ref[...], preferred_element_type=jnp.float32)
    h2p_ref[...] = h2 * dinv


def _tc_c(agg_a_ref, agg_b_ref, h2p_ref, degc_ref, b2_ref, out_ref):
    dinv = _dinv_col(degc_ref)
    aggsum = agg_a_ref[...] + agg_b_ref[...]
    o = (dinv * (aggsum + h2p_ref[...]) + b2_ref[...])[0:N, :]
    m = jnp.max(o, axis=1, keepdims=True)
    e = jnp.exp(o - m)
    lse = jnp.log(jnp.sum(e, axis=1, keepdims=True))
    out_ref[...] = o - m - lse


def _tc_call(body, out_shape, *args):
    return pl.pallas_call(
        body,
        out_shape=jax.ShapeDtypeStruct(out_shape, jnp.float32),
    )(*args)


def kernel(x, edge_index, W1, b1, bn_w, bn_b, W2, b2):
    f32 = jnp.float32
    pad_idx = (jnp.arange(EP - E, dtype=jnp.int32) % (NP - N)) + N
    src = jnp.reshape(jnp.concatenate([edge_index[0], pad_idx]), (EP // C, C))
    dst = jnp.reshape(jnp.concatenate([edge_index[1], pad_idx]), (EP // C, C))
    zeros_n = jnp.zeros((NP,), f32)
    zeros_32 = jnp.zeros((NP, 32), f32)
    zeros_64 = jnp.zeros((NP, 64), f32)

    dega, degb = _sc_degree(dst, zeros_n)
    degcols = jnp.stack([dega, degb], axis=1)

    h1p = _tc_call(_tc_a, (NP, 32), x, W1, degcols)

    agg1a, agg1b = _sc_agg32(h1p, src, dst, zeros_32)

    h2p = _tc_call(_tc_b, (NP, 64),
                   agg1a, agg1b, h1p, degcols,
                   jnp.reshape(b1, (1, 32)), jnp.reshape(bn_w, (1, 32)),
                   jnp.reshape(bn_b, (1, 32)), W2)

    agg2a, agg2b = _sc_agg64(h2p, src, dst, zeros_64)

    return _tc_call(_tc_c, (N, 64),
                    agg2a, agg2b, h2p, degcols,
                    jnp.reshape(b2, (1, 64)))


# speedup vs baseline: 1.0380x; 1.0059x over previous
import functools

import jax
import jax.numpy as jnp
from jax import lax
from jax.experimental import pallas as pl
from jax.experimental.pallas import tpu as pltpu
from jax.experimental.pallas import tpu_sc as plsc

N = 10000
E = 320000
NP = 10240
NT = 16
NC = 2
NW = NC * NT
RPT = NP // NT
C = 128
CPW = 80
EPW = CPW * C
EP = NW * EPW


def _sc_mesh():
    return plsc.VectorSubcoreMesh(core_axis_name="c", subcore_axis_name="s")


_SC_PARAMS = pltpu.CompilerParams(use_tc_tiling_on_sc=False)


_DEG_K = 16

@functools.partial(
    pl.kernel,
    mesh=_sc_mesh(),
    compiler_params=_SC_PARAMS,
    out_type=[jax.ShapeDtypeStruct((NP,), jnp.float32),
              jax.ShapeDtypeStruct((NP,), jnp.float32)],
    scratch_types=[
        pltpu.VMEM((_DEG_K, C), jnp.int32),
        pltpu.VMEM((C,), jnp.float32),
        pltpu.VMEM_SHARED((NP,), jnp.float32),
        pltpu.SemaphoreType.DMA,
        pltpu.SemaphoreType.DMA,
    ],
)
def _sc_degree(dst2_hbm, zeros_hbm, out_a_hbm, out_b_hbm, didx, ones_v, acc,
               isem, ssem):
    cid = lax.axis_index("c")
    sid = lax.axis_index("s")
    wid = cid * NT + sid
    r0 = sid * RPT
    for i in range(C // 16):
        ones_v[pl.ds(i * 16, 16)] = jnp.ones((16,), jnp.float32)
    pltpu.sync_copy(zeros_hbm.at[pl.ds(r0, RPT)], acc.at[pl.ds(r0, RPT)])
    plsc.subcore_barrier()

    def body(blk, carry):
        row0 = pl.multiple_of(wid * CPW + blk * _DEG_K, _DEG_K)
        pltpu.async_copy(dst2_hbm.at[pl.ds(row0, _DEG_K)], didx, isem).wait()
        ss = [pltpu.async_copy(ones_v, acc.at[didx.at[k]], ssem, add=True)
              for k in range(_DEG_K)]
        for s in ss:
            s.wait()
        return carry

    lax.fori_loop(0, CPW // _DEG_K, body, 0)
    plsc.subcore_barrier()
    @pl.when(cid == 0)
    def _():
        pltpu.sync_copy(acc.at[pl.ds(r0, RPT)], out_a_hbm.at[pl.ds(r0, RPT)])
    @pl.when(cid == 1)
    def _():
        pltpu.sync_copy(acc.at[pl.ds(r0, RPT)], out_b_hbm.at[pl.ds(r0, RPT)])


def _make_sc_agg(F, K):
    NBLK = CPW // K

    @functools.partial(
        pl.kernel,
        mesh=_sc_mesh(),
        compiler_params=_SC_PARAMS,
        out_type=[jax.ShapeDtypeStruct((NP, F), jnp.float32),
                  jax.ShapeDtypeStruct((NP, F), jnp.float32)],
        scratch_types=[
            pltpu.VMEM((3, K, C), jnp.int32),
            pltpu.VMEM((3, K, C), jnp.int32),
            pltpu.VMEM((2, K, C, F), jnp.float32),
            pltpu.VMEM_SHARED((NP, F), jnp.float32),
            pltpu.SemaphoreType.DMA,
            pltpu.SemaphoreType.DMA,
            pltpu.SemaphoreType.DMA,
            pltpu.SemaphoreType.DMA,
            pltpu.SemaphoreType.DMA,
        ],
    )
    def _sc_agg(h_hbm, src2_hbm, dst2_hbm, zeros_hbm, out_a_hbm, out_b_hbm,
                sidx, didx, rows, acc, is0, is1, is2, gsem, ssem):
        cid = lax.axis_index("c")
        sid = lax.axis_index("s")
        wid = cid * NT + sid
        r0 = sid * RPT
        pltpu.sync_copy(zeros_hbm.at[pl.ds(r0, RPT)], acc.at[pl.ds(r0, RPT)])
        plsc.subcore_barrier()
        base = wid * CPW
        isems = [is0, is1, is2]

        def fire_idx(b):
            sl = b % 3
            r = pl.multiple_of(base + b * K, K)
            return (pltpu.async_copy(src2_hbm.at[pl.ds(r, K)], sidx.at[sl],
                                     isems[sl]),
                    pltpu.async_copy(dst2_hbm.at[pl.ds(r, K)], didx.at[sl],
                                     isems[sl]))

        def fire_g(b):
            return [pltpu.async_copy(h_hbm.at[sidx.at[b % 3, k]],
                                     rows.at[b % 2, k], gsem)
                    for k in range(K)]

        def fire_s(b):
            return [pltpu.async_copy(rows.at[b % 2, k],
                                     acc.at[didx.at[b % 3, k]], ssem,
                                     add=True)
                    for k in range(K)]

        idxh, gh, sh = {}, {}, {}
        for b in range(min(3, NBLK)):
            idxh[b] = fire_idx(b)
        for h in idxh[0]:
            h.wait()
        gh[0] = fire_g(0)
        for h in gh[0]:
            h.wait()
        sh[0] = fire_s(0)
        for h in idxh[1]:
            h.wait()
        gh[1] = fire_g(1)
        for b in range(2, NBLK):
            for h in sh[b - 2]:
                h.wait()
            if b + 1 < NBLK:
                idxh[b + 1] = fire_idx(b + 1)
            for h in gh[b - 1]:
                h.wait()
            sh[b - 1] = fire_s(b - 1)
            for h in idxh[b]:
                h.wait()
            gh[b] = fire_g(b)
        for h in gh[NBLK - 1]:
            h.wait()
        sh[NBLK - 1] = fire_s(NBLK - 1)
        for h in sh[NBLK - 2]:
            h.wait()
        for h in sh[NBLK - 1]:
            h.wait()
        plsc.subcore_barrier()
        @pl.when(cid == 0)
        def _():
            pltpu.sync_copy(acc.at[pl.ds(r0, RPT)],
                            out_a_hbm.at[pl.ds(r0, RPT)])
        @pl.when(cid == 1)
        def _():
            pltpu.sync_copy(acc.at[pl.ds(r0, RPT)],
                            out_b_hbm.at[pl.ds(r0, RPT)])

    return _sc_agg


_sc_agg32 = _make_sc_agg(32, 10)
_sc_agg64 = _make_sc_agg(64, 5)


def _dinv_col(degcols_ref):
    deg = degcols_ref[:, 0:1] + degcols_ref[:, 1:2] + 1.0
    dinv = lax.rsqrt(deg)
    rows = lax.broadcasted_iota(jnp.int32, (NP, 1), 0)
    return jnp.where(rows < N, dinv, 0.0)


def _tc_a0(x_ref, w1_ref, h1_ref):
    h1_ref[...] = jnp.dot(x_ref[...], w1_ref[...],
                          preferred_element_type=jnp.float32)


def _tc_a1(h1_ref, degc_ref, h1p_ref):
    dinv = _dinv_col(degc_ref)
    h1p_ref[0:N, :] = h1_ref[...] * dinv[0:N, :]
    h1p_ref[N:NP, :] = jnp.zeros((NP - N, 32), jnp.float32)


def _tc_b(agg_a_ref, agg_b_ref, h1p_ref, degc_ref, b1_ref, bnw_ref, bnb_ref,
          w2_ref, h2p_ref):
    dinv = _dinv_col(degc_ref)
    aggsum = agg_a_ref[...] + agg_b_ref[...]
    out1 = dinv * (aggsum + h1p_ref[...]) + b1_ref[...]
    rows = lax.broadcasted_iota(jnp.int32, (NP, 1), 0)
    mask = rows < N
    inv_n = jnp.float32(1.0 / N)
    mean = jnp.sum(jnp.where(mask, out1, 0.0), axis=0, keepdims=True) * inv_n
    cent = out1 - mean
    var = jnp.sum(jnp.where(mask, cent * cent, 0.0), axis=0,
                  keepdims=True) * inv_n
    z = cent * lax.rsqrt(var + 1e-5) * bnw_ref[...] + bnb_ref[...]
    z = jnp.maximum(z, 0.0)
    h2 = jnp.dot(z, w2_ref[...], preferred_element_type=jnp.float32)
    h2p_ref[...] = h2 * dinv


def _tc_c(agg_a_ref, agg_b_ref, h2p_ref, degc_ref, b2_ref, out_ref):
    dinv = _dinv_col(degc_ref)
    aggsum = agg_a_ref[...] + agg_b_ref[...]
    o = (dinv * (aggsum + h2p_ref[...]) + b2_ref[...])[0:N, :]
    m = jnp.max(o, axis=1, keepdims=True)
    e = jnp.exp(o - m)
    lse = jnp.log(jnp.sum(e, axis=1, keepdims=True))
    out_ref[...] = o - m - lse


def _tc_call(body, out_shape, *args):
    return pl.pallas_call(
        body,
        out_shape=jax.ShapeDtypeStruct(out_shape, jnp.float32),
    )(*args)


def kernel(x, edge_index, W1, b1, bn_w, bn_b, W2, b2):
    f32 = jnp.float32
    pad_idx = (jnp.arange(EP - E, dtype=jnp.int32) % (NP - N)) + N
    src = jnp.reshape(jnp.concatenate([edge_index[0], pad_idx]), (EP // C, C))
    dst = jnp.reshape(jnp.concatenate([edge_index[1], pad_idx]), (EP // C, C))
    zeros_n = jnp.zeros((NP,), f32)
    zeros_32 = jnp.zeros((NP, 32), f32)
    zeros_64 = jnp.zeros((NP, 64), f32)

    dega, degb = _sc_degree(dst, zeros_n)
    h1 = _tc_call(_tc_a0, (N, 32), x, W1)
    degcols = jnp.stack([dega, degb], axis=1)

    h1p = _tc_call(_tc_a1, (NP, 32), h1, degcols)

    agg1a, agg1b = _sc_agg32(h1p, src, dst, zeros_32)

    h2p = _tc_call(_tc_b, (NP, 64),
                   agg1a, agg1b, h1p, degcols,
                   jnp.reshape(b1, (1, 32)), jnp.reshape(bn_w, (1, 32)),
                   jnp.reshape(bn_b, (1, 32)), W2)

    agg2a, agg2b = _sc_agg64(h2p, src, dst, zeros_64)

    return _tc_call(_tc_c, (N, 64),
                    agg2a, agg2b, h2p, degcols,
                    jnp.reshape(b2, (1, 64)))
